# baseline (speedup 1.0000x reference)
import functools

import jax
import jax.numpy as jnp
from jax import lax
from jax.experimental import pallas as pl
from jax.experimental.pallas import tpu as pltpu
from jax.experimental.pallas import tpu_sc as plsc

NC = 2
NS = 16
L = 16
NW = NC * NS
B = 16384
D = 32
BPW = B // NW
CH = 128
NCH = BPW // CH


def _mf_body(uid_h, iid_h, uemb_h, iemb_h, ubias_h, ibias_h, mean_h, out_h,
             uidx, iidx, urows, irows, ub, ib, ov, mv, sem):
    c = lax.axis_index("c")
    s = lax.axis_index("s")
    wid = s * NC + c
    base = wid * BPW

    for j in range(NCH):
        pltpu.sync_copy(uid_h.at[pl.ds(base + j * CH, CH)], uidx.at[j])
        pltpu.sync_copy(iid_h.at[pl.ds(base + j * CH, CH)], iidx.at[j])
    pltpu.sync_copy(mean_h, mv)

    copies = []
    for j in range(NCH):
        sl = pl.ds(j * CH, CH)
        copies.append(pltpu.async_copy(uemb_h.at[uidx.at[j]], urows.at[sl], sem))
        copies.append(pltpu.async_copy(iemb_h.at[iidx.at[j]], irows.at[sl], sem))
        copies.append(pltpu.async_copy(ubias_h.at[uidx.at[j]], ub.at[sl], sem))
        copies.append(pltpu.async_copy(ibias_h.at[iidx.at[j]], ib.at[sl], sem))
    for cp in copies:
        cp.wait()

    mean_vec = mv[...]
    lane = lax.iota(jnp.int32, L)

    def group(g, acc0):
        sl = pl.ds(g * L, L)
        acc = ub[sl] + ib[sl] + mean_vec
        for k in range(L):
            s = g * L + k
            u0 = urows[s, pl.ds(0, L)]
            u1 = urows[s, pl.ds(L, L)]
            i0 = irows[s, pl.ds(0, L)]
            i1 = irows[s, pl.ds(L, L)]
            t = jnp.sum(u0 * i0 + u1 * i1)
            acc = jnp.where(lane == k, acc + t, acc)
        ov[sl] = acc
        return acc0

    lax.fori_loop(0, BPW // L, group, 0)
    pltpu.sync_copy(ov, out_h.at[pl.ds(base, BPW)])


@jax.jit
def kernel(u_id, i_id, user_emb, item_emb, user_bias, item_bias, mean):
    mesh = plsc.VectorSubcoreMesh(core_axis_name="c", subcore_axis_name="s")
    f = pl.kernel(
        _mf_body,
        mesh=mesh,
        compiler_params=pltpu.CompilerParams(
            needs_layout_passes=False, use_tc_tiling_on_sc=False),
        out_type=jax.ShapeDtypeStruct((B,), jnp.float32),
        scratch_types=[
            pltpu.VMEM((NCH, CH), jnp.int32),
            pltpu.VMEM((NCH, CH), jnp.int32),
            pltpu.VMEM((BPW, D), jnp.float32),
            pltpu.VMEM((BPW, D), jnp.float32),
            pltpu.VMEM((BPW,), jnp.float32),
            pltpu.VMEM((BPW,), jnp.float32),
            pltpu.VMEM((BPW,), jnp.float32),
            pltpu.VMEM((L,), jnp.float32),
            pltpu.SemaphoreType.DMA,
        ],
    )
    return f(u_id, i_id, user_emb, item_emb,
             user_bias.reshape(-1), item_bias.reshape(-1),
             jnp.broadcast_to(mean, (L,)))
